# baseline (speedup 1.0000x reference)
import jax
import jax.numpy as jnp
from jax import lax
from jax.experimental import pallas as pl
from jax.experimental.pallas import tpu as pltpu
from jax.experimental.pallas import tpu_sc as plsc

E = 320000
V = 10000
D = 128
F = 16
W = 2 * D + F
CHUNK = 128
NLONG = 17
LMAX = 80 * CHUNK
LMIN = 76 * CHUNK


def _join_body(unary, binary_r, idx1, idx2, out, idx1_v, idx2_v, joined,
               bin_v, gsem, hsem, ssem):
    c = lax.axis_index("c")
    s = lax.axis_index("s")
    w = 16 * c + s
    a0 = pl.multiple_of(512 * (19 * w + jnp.minimum(w, NLONG)), 512)
    ncw = jnp.where(w < NLONG, 80, 76)

    @pl.when(w < NLONG)
    def _():
        pltpu.sync_copy(idx1.at[pl.ds(a0, LMAX)], idx1_v)
        pltpu.sync_copy(idx2.at[pl.ds(a0, LMAX)], idx2_v)

    @pl.when(w >= NLONG)
    def _():
        pltpu.sync_copy(idx1.at[pl.ds(a0, LMIN)], idx1_v.at[pl.ds(0, LMIN)])
        pltpu.sync_copy(idx2.at[pl.ds(a0, LMIN)], idx2_v.at[pl.ds(0, LMIN)])

    def start_bin(slot, i):
        rb = pl.multiple_of(a0 // 8 + 16 * i, 8)
        pltpu.async_copy(binary_r.at[pl.ds(rb, 16)], bin_v.at[slot],
                         hsem.at[slot])

    def wait_bin(slot):
        pltpu.make_async_copy(binary_r.at[pl.ds(0, 16)], bin_v.at[slot],
                              hsem.at[slot]).wait()

    def start_gathers(slot, i):
        base = pl.multiple_of(i * CHUNK, 8)
        pltpu.async_copy(unary.at[idx1_v.at[pl.ds(base, CHUNK)]],
                         joined.at[slot, :, pl.ds(0, D)], gsem.at[slot])
        pltpu.async_copy(unary.at[idx2_v.at[pl.ds(base, CHUNK)]],
                         joined.at[slot, :, pl.ds(D, D)], gsem.at[slot])

    def wait_gathers(slot):
        pltpu.make_async_copy(unary.at[idx1_v.at[pl.ds(0, CHUNK)]],
                              joined.at[slot, :, pl.ds(0, D)],
                              gsem.at[slot]).wait()
        pltpu.make_async_copy(unary.at[idx2_v.at[pl.ds(0, CHUNK)]],
                              joined.at[slot, :, pl.ds(D, D)],
                              gsem.at[slot]).wait()

    def start_out(slot, i):
        g = pl.multiple_of(a0 + i * CHUNK, 8)
        pltpu.async_copy(joined.at[slot], out.at[pl.ds(g, CHUNK), :],
                         ssem.at[slot])

    def wait_out(slot):
        pltpu.make_async_copy(joined.at[slot], out.at[pl.ds(0, CHUNK), :],
                              ssem.at[slot]).wait()

    start_bin(0, 0)
    start_bin(1, 1)

    def do_chunk(slot, i):
        @pl.when((i >= 2) & (i - 2 < ncw))
        def _():
            wait_out(slot)

        @pl.when(i < ncw)
        def _():
            start_gathers(slot, i)
            wait_bin(slot)

        for e in range(CHUNK):
            joined[slot, e, pl.ds(2 * D, F)] = (
                bin_v[slot, e // 8, pl.ds((e % 8) * F, F)])

        @pl.when(i + 2 < ncw)
        def _():
            start_bin(slot, i + 2)

        @pl.when(i < ncw)
        def _():
            wait_gathers(slot)
            start_out(slot, i)

    def pair(kp, carry):
        do_chunk(0, 2 * kp)
        do_chunk(1, 2 * kp + 1)
        return carry

    lax.fori_loop(0, 40, pair, 0)

    @pl.when(ncw == 80)
    def _():
        wait_out(0)
        wait_out(1)


def kernel(unary, binary, index1, index2):
    mesh = plsc.VectorSubcoreMesh(core_axis_name="c", subcore_axis_name="s")
    f = pl.kernel(
        _join_body,
        mesh=mesh,
        out_type=jax.ShapeDtypeStruct((E, W), jnp.float32),
        scratch_types=[
            pltpu.VMEM((LMAX,), jnp.int32),
            pltpu.VMEM((LMAX,), jnp.int32),
            pltpu.VMEM((2, CHUNK, W), jnp.float32),
            pltpu.VMEM((2, 16, D), jnp.float32),
            pltpu.SemaphoreType.DMA((2,)),
            pltpu.SemaphoreType.DMA((2,)),
            pltpu.SemaphoreType.DMA((2,)),
        ],
    )
    return f(unary, binary.reshape(V * 4, D), index1.astype(jnp.int32),
             index2.astype(jnp.int32))


# speedup vs baseline: 1.0029x; 1.0029x over previous
import jax
import jax.numpy as jnp
from jax import lax
from jax.experimental import pallas as pl
from jax.experimental.pallas import tpu as pltpu
from jax.experimental.pallas import tpu_sc as plsc

E = 320000
V = 10000
D = 128
F = 16
W = 2 * D + F
CHUNK = 128
NLONG = 17
LMAX = 80 * CHUNK
LMIN = 76 * CHUNK


def _join_body(unary, binary, idx1, idx2, out, idx1_v, idx2_v, joined,
               gsem, ssem):
    c = lax.axis_index("c")
    s = lax.axis_index("s")
    w = 16 * c + s
    a0 = pl.multiple_of(512 * (19 * w + jnp.minimum(w, NLONG)), 512)
    ncw = jnp.where(w < NLONG, 80, 76)

    @pl.when(w < NLONG)
    def _():
        pltpu.sync_copy(idx1.at[pl.ds(a0, LMAX)], idx1_v)
        pltpu.sync_copy(idx2.at[pl.ds(a0, LMAX)], idx2_v)

    @pl.when(w >= NLONG)
    def _():
        pltpu.sync_copy(idx1.at[pl.ds(a0, LMIN)], idx1_v.at[pl.ds(0, LMIN)])
        pltpu.sync_copy(idx2.at[pl.ds(a0, LMIN)], idx2_v.at[pl.ds(0, LMIN)])

    def start_in(slot, i):
        base = pl.multiple_of(i * CHUNK, 8)
        g = pl.multiple_of(a0 + i * CHUNK, 8)
        pltpu.async_copy(unary.at[idx1_v.at[pl.ds(base, CHUNK)]],
                         joined.at[slot, :, pl.ds(0, D)], gsem.at[slot])
        pltpu.async_copy(unary.at[idx2_v.at[pl.ds(base, CHUNK)]],
                         joined.at[slot, :, pl.ds(D, D)], gsem.at[slot])
        pltpu.async_copy(binary.at[pl.ds(g, CHUNK), :],
                         joined.at[slot, :, pl.ds(2 * D, F)], gsem.at[slot])

    def wait_in(slot):
        pltpu.make_async_copy(unary.at[idx1_v.at[pl.ds(0, CHUNK)]],
                              joined.at[slot, :, pl.ds(0, D)],
                              gsem.at[slot]).wait()
        pltpu.make_async_copy(unary.at[idx2_v.at[pl.ds(0, CHUNK)]],
                              joined.at[slot, :, pl.ds(D, D)],
                              gsem.at[slot]).wait()
        pltpu.make_async_copy(binary.at[pl.ds(0, CHUNK), :],
                              joined.at[slot, :, pl.ds(2 * D, F)],
                              gsem.at[slot]).wait()

    def start_out(slot, i):
        g = pl.multiple_of(a0 + i * CHUNK, 8)
        pltpu.async_copy(joined.at[slot], out.at[pl.ds(g, CHUNK), :],
                         ssem.at[slot])

    def wait_out(slot):
        pltpu.make_async_copy(joined.at[slot], out.at[pl.ds(0, CHUNK), :],
                              ssem.at[slot]).wait()

    def do_chunk(slot, i):
        @pl.when((i >= 2) & (i - 2 < ncw))
        def _():
            wait_out(slot)

        @pl.when(i < ncw)
        def _():
            start_in(slot, i)
            wait_in(slot)
            start_out(slot, i)

    def pair(kp, carry):
        do_chunk(0, 2 * kp)
        do_chunk(1, 2 * kp + 1)
        return carry

    lax.fori_loop(0, 40, pair, 0)

    @pl.when(ncw == 80)
    def _():
        wait_out(0)
        wait_out(1)


def kernel(unary, binary, index1, index2):
    mesh = plsc.VectorSubcoreMesh(core_axis_name="c", subcore_axis_name="s")
    f = pl.kernel(
        _join_body,
        mesh=mesh,
        out_type=jax.ShapeDtypeStruct((E, W), jnp.float32),
        scratch_types=[
            pltpu.VMEM((LMAX,), jnp.int32),
            pltpu.VMEM((LMAX,), jnp.int32),
            pltpu.VMEM((2, CHUNK, W), jnp.float32),
            pltpu.SemaphoreType.DMA((2,)),
            pltpu.SemaphoreType.DMA((2,)),
        ],
    )
    return f(unary, binary, index1.astype(jnp.int32), index2.astype(jnp.int32))


# speedup vs baseline: 1.0100x; 1.0071x over previous
import jax
import jax.numpy as jnp
from jax import lax
from jax.experimental import pallas as pl
from jax.experimental.pallas import tpu as pltpu
from jax.experimental.pallas import tpu_sc as plsc

E = 320000
V = 10000
D = 128
F = 16
W = 2 * D + F
CHUNK = 128
NLONG = 17
LMAX = 80 * CHUNK
LMIN = 76 * CHUNK


def _join_body(unary, binary, idx1, idx2, out, idx1_v, idx2_v, joined,
               gsem, ssem):
    c = lax.axis_index("c")
    s = lax.axis_index("s")
    w = 16 * c + s
    a0 = pl.multiple_of(512 * (19 * w + jnp.minimum(w, NLONG)), 512)
    ncw = jnp.where(w < NLONG, 80, 76)

    @pl.when(w < NLONG)
    def _():
        pltpu.sync_copy(idx1.at[pl.ds(a0, LMAX)], idx1_v)
        pltpu.sync_copy(idx2.at[pl.ds(a0, LMAX)], idx2_v)

    @pl.when(w >= NLONG)
    def _():
        pltpu.sync_copy(idx1.at[pl.ds(a0, LMIN)], idx1_v.at[pl.ds(0, LMIN)])
        pltpu.sync_copy(idx2.at[pl.ds(a0, LMIN)], idx2_v.at[pl.ds(0, LMIN)])

    def start_in(slot, i):
        base = pl.multiple_of(i * CHUNK, 8)
        g = pl.multiple_of(a0 + i * CHUNK, 8)
        pltpu.async_copy(unary.at[idx1_v.at[pl.ds(base, CHUNK)]],
                         joined.at[slot, :, pl.ds(0, D)], gsem.at[slot])
        pltpu.async_copy(unary.at[idx2_v.at[pl.ds(base, CHUNK)]],
                         joined.at[slot, :, pl.ds(D, D)], gsem.at[slot])
        pltpu.async_copy(binary.at[pl.ds(g, CHUNK), :],
                         joined.at[slot, :, pl.ds(2 * D, F)], gsem.at[slot])

    def wait_in(slot):
        pltpu.make_async_copy(unary.at[idx1_v.at[pl.ds(0, CHUNK)]],
                              joined.at[slot, :, pl.ds(0, D)],
                              gsem.at[slot]).wait()
        pltpu.make_async_copy(unary.at[idx2_v.at[pl.ds(0, CHUNK)]],
                              joined.at[slot, :, pl.ds(D, D)],
                              gsem.at[slot]).wait()
        pltpu.make_async_copy(binary.at[pl.ds(0, CHUNK), :],
                              joined.at[slot, :, pl.ds(2 * D, F)],
                              gsem.at[slot]).wait()

    def start_out(slot, i):
        g = pl.multiple_of(a0 + i * CHUNK, 8)
        pltpu.async_copy(joined.at[slot], out.at[pl.ds(g, CHUNK), :],
                         ssem.at[slot])

    def wait_out(slot):
        pltpu.make_async_copy(joined.at[slot], out.at[pl.ds(0, CHUNK), :],
                              ssem.at[slot]).wait()

    start_in(0, 0)

    def do_chunk(slot, i):
        nslot = 1 - slot

        @pl.when(i + 1 < ncw)
        def _():
            @pl.when(i >= 1)
            def _():
                wait_out(nslot)
            start_in(nslot, i + 1)

        @pl.when(i < ncw)
        def _():
            wait_in(slot)
            start_out(slot, i)

    def pair(kp, carry):
        do_chunk(0, 2 * kp)
        do_chunk(1, 2 * kp + 1)
        return carry

    lax.fori_loop(0, 40, pair, 0)
    wait_out(0)
    wait_out(1)


def kernel(unary, binary, index1, index2):
    mesh = plsc.VectorSubcoreMesh(core_axis_name="c", subcore_axis_name="s")
    f = pl.kernel(
        _join_body,
        mesh=mesh,
        out_type=jax.ShapeDtypeStruct((E, W), jnp.float32),
        scratch_types=[
            pltpu.VMEM((LMAX,), jnp.int32),
            pltpu.VMEM((LMAX,), jnp.int32),
            pltpu.VMEM((2, CHUNK, W), jnp.float32),
            pltpu.SemaphoreType.DMA((2,)),
            pltpu.SemaphoreType.DMA((2,)),
        ],
    )
    return f(unary, binary, index1.astype(jnp.int32), index2.astype(jnp.int32))


# speedup vs baseline: 1.0168x; 1.0068x over previous
import jax
import jax.numpy as jnp
from jax import lax
from jax.experimental import pallas as pl
from jax.experimental.pallas import tpu as pltpu
from jax.experimental.pallas import tpu_sc as plsc

E = 320000
V = 10000
D = 128
F = 16
W = 2 * D + F
NW = 32
PER_W = E // NW
CHUNK = 80
NCH = PER_W // CHUNK
NBUF = 3


def _join_body(unary, binary, idx1, idx2, out, idx1_v, idx2_v, joined, gsem,
               ssem):
    c = lax.axis_index("c")
    s = lax.axis_index("s")
    wid = s * 2 + c
    w0 = pl.multiple_of(wid * PER_W, 8)
    pltpu.sync_copy(idx1.at[pl.ds(w0, PER_W)], idx1_v)
    pltpu.sync_copy(idx2.at[pl.ds(w0, PER_W)], idx2_v)

    def start_in(slot, i):
        base = pl.multiple_of(i * CHUNK, 8)
        g = pl.multiple_of(w0 + base, 8)
        pltpu.async_copy(unary.at[idx1_v.at[pl.ds(base, CHUNK)]],
                         joined.at[slot, :, pl.ds(0, D)], gsem.at[slot])
        pltpu.async_copy(unary.at[idx2_v.at[pl.ds(base, CHUNK)]],
                         joined.at[slot, :, pl.ds(D, D)], gsem.at[slot])
        pltpu.async_copy(binary.at[pl.ds(g, CHUNK), :],
                         joined.at[slot, :, pl.ds(2 * D, F)], gsem.at[slot])

    def wait_in(slot):
        pltpu.make_async_copy(unary.at[idx1_v.at[pl.ds(0, CHUNK)]],
                              joined.at[slot, :, pl.ds(0, D)],
                              gsem.at[slot]).wait()
        pltpu.make_async_copy(unary.at[idx2_v.at[pl.ds(0, CHUNK)]],
                              joined.at[slot, :, pl.ds(D, D)],
                              gsem.at[slot]).wait()
        pltpu.make_async_copy(binary.at[pl.ds(0, CHUNK), :],
                              joined.at[slot, :, pl.ds(2 * D, F)],
                              gsem.at[slot]).wait()

    def start_out(slot, i):
        g = pl.multiple_of(w0 + i * CHUNK, 8)
        pltpu.async_copy(joined.at[slot], out.at[pl.ds(g, CHUNK), :],
                         ssem.at[slot])

    def wait_out(slot):
        pltpu.make_async_copy(joined.at[slot], out.at[pl.ds(w0, CHUNK), :],
                              ssem.at[slot]).wait()

    for k in range(NBUF - 1):
        start_in(k, k)

    def body(i, carry):
        slot = lax.rem(i, NBUF)
        pre = lax.rem(i + NBUF - 1, NBUF)

        @pl.when(i + NBUF - 1 < NCH)
        def _():
            @pl.when(i >= 1)
            def _():
                wait_out(pre)
            start_in(pre, i + NBUF - 1)

        wait_in(slot)
        start_out(slot, i)
        return carry

    lax.fori_loop(0, NCH, body, 0)
    for k in range(NBUF):
        wait_out((NCH - NBUF + k) % NBUF)


def kernel(unary, binary, index1, index2):
    mesh = plsc.VectorSubcoreMesh(core_axis_name="c", subcore_axis_name="s")
    f = pl.kernel(
        _join_body,
        mesh=mesh,
        out_type=jax.ShapeDtypeStruct((E, W), jnp.float32),
        scratch_types=[
            pltpu.VMEM((PER_W,), jnp.int32),
            pltpu.VMEM((PER_W,), jnp.int32),
            pltpu.VMEM((NBUF, CHUNK, W), jnp.float32),
            pltpu.SemaphoreType.DMA((NBUF,)),
            pltpu.SemaphoreType.DMA((NBUF,)),
        ],
    )
    return f(unary, binary, index1.astype(jnp.int32), index2.astype(jnp.int32))
